# baseline (speedup 1.0000x reference)
import functools

import jax
import jax.numpy as jnp
from jax.experimental import pallas as pl
from jax.experimental.pallas import tpu as pltpu

B, H, L, E = 32, 16, 192, 3
BH = B * H
TOP_P = 0.5
NOISE_EPS = 0.01
EPS_C = 1e-10
BLK = 8
GRID = BH // BLK


def _moe_body(x_ref, eps_ref, wg_ref, wn_ref, m_ref, out_ref, loss_ref,
              s0_ref, s1_ref, ent_ref):
    step = pl.program_id(0)

    @pl.when(step == 0)
    def _init():
        s0_ref[...] = jnp.zeros_like(s0_ref)
        s1_ref[...] = jnp.zeros_like(s1_ref)
        ent_ref[...] = jnp.zeros_like(ent_ref)

    m0 = m_ref[0]
    m1 = m_ref[1]
    m2 = m_ref[2]
    rows = jax.lax.broadcasted_iota(jnp.int32, (L, L), 0)
    cols = jax.lax.broadcasted_iota(jnp.int32, (L, L), 1)
    eye = jnp.where(rows == cols, 1.0, 0.0)

    s0_tot = jnp.zeros((L, 1), jnp.float32)
    s1_tot = jnp.zeros((L, 1), jnp.float32)
    ent_tot = jnp.zeros((L, 1), jnp.float32)
    zero = jnp.zeros((L, 1), jnp.float32)

    for i in range(BLK):
        a = x_ref[i]
        clean = jnp.dot(a, wg_ref[...], preferred_element_type=jnp.float32)
        rawn = jnp.dot(a, wn_ref[...], preferred_element_type=jnp.float32)
        noisy = clean + eps_ref[i] * (jax.nn.softplus(rawn) + NOISE_EPS)
        mx = jnp.max(noisy, axis=1, keepdims=True)
        ex = jnp.exp(noisy - mx)
        p = ex / jnp.sum(ex, axis=1, keepdims=True)
        ent_tot += jnp.sum(p * jnp.log(p + EPS_C), axis=1, keepdims=True)

        p0 = p[:, 0:1]
        p1 = p[:, 1:2]
        p2 = p[:, 2:3]
        a10 = p1 > p0
        a20 = p2 > p0
        a01 = p0 >= p1
        a21 = p2 > p1
        a02 = p0 >= p2
        a12 = p1 >= p2
        cb0 = jnp.where(a10, p1, zero) + jnp.where(a20, p2, zero)
        cb1 = jnp.where(a01, p0, zero) + jnp.where(a21, p2, zero)
        cb2 = jnp.where(a02, p0, zero) + jnp.where(a12, p1, zero)
        k0 = cb0 <= TOP_P
        k1 = cb1 <= TOP_P
        k2 = cb2 <= TOP_P
        r0 = a10.astype(jnp.int32) + a20.astype(jnp.int32)
        r1 = a01.astype(jnp.int32) + a21.astype(jnp.int32)
        r2 = a02.astype(jnp.int32) + a12.astype(jnp.int32)
        s0_tot += (jnp.where(r0 == 0, p0, zero) + jnp.where(r1 == 0, p1, zero)
                   + jnp.where(r2 == 0, p2, zero))
        s1_tot += (jnp.where((r0 == 1) & k0, p0, zero)
                   + jnp.where((r1 == 1) & k1, p1, zero)
                   + jnp.where((r2 == 1) & k2, p2, zero))
        out_ref[i] = (jnp.where(k0, m0, 0.0) + jnp.where(k1, m1, 0.0)
                      + jnp.where(k2, m2, 0.0) + eye)

    s0_ref[...] += s0_tot
    s1_ref[...] += s1_tot
    ent_ref[...] += ent_tot

    @pl.when(step == GRID - 1)
    def _finalize():
        s0 = s0_ref[...]
        s1 = s1_ref[...]
        n = float(L * E)
        tot = jnp.sum(s0) + jnp.sum(s1)
        sq = jnp.sum(s0 * s0) + jnp.sum(s1 * s1)
        mean = tot / n
        var = (sq - n * mean * mean) / (n - 1.0)
        loss_imp = var / (mean * mean + EPS_C)
        loss_dyn = -jnp.sum(ent_ref[...]) / float(BH * E)
        loss_ref[...] = jnp.reshape(loss_imp + 0.1 * loss_dyn, (1, 1))


@functools.partial(jax.jit, static_argnames=())
def kernel(x, masks, W_gate, W_noise):
    xf = x.reshape(BH, L, L)
    eps = jax.random.normal(jax.random.key(42), (BH, L, E), dtype=jnp.float32)
    masks_t = jnp.transpose(masks, (1, 0, 2))
    out, loss = pl.pallas_call(
        _moe_body,
        grid=(GRID,),
        in_specs=[
            pl.BlockSpec((BLK, L, L), lambda i: (i, 0, 0)),
            pl.BlockSpec((BLK, L, E), lambda i: (i, 0, 0)),
            pl.BlockSpec((L, E), lambda i: (0, 0)),
            pl.BlockSpec((L, E), lambda i: (0, 0)),
            pl.BlockSpec((E, L, L), lambda i: (0, 0, 0)),
        ],
        out_specs=[
            pl.BlockSpec((BLK, L, L), lambda i: (i, 0, 0)),
            pl.BlockSpec((1, 1), lambda i: (0, 0)),
        ],
        out_shape=[
            jax.ShapeDtypeStruct((BH, L, L), jnp.float32),
            jax.ShapeDtypeStruct((1, 1), jnp.float32),
        ],
        scratch_shapes=[
            pltpu.VMEM((L, 1), jnp.float32),
            pltpu.VMEM((L, 1), jnp.float32),
            pltpu.VMEM((L, 1), jnp.float32),
        ],
        compiler_params=pltpu.CompilerParams(
            dimension_semantics=("arbitrary",),
        ),
    )(xf, eps, W_gate.T, W_noise.T, masks_t)
    return out.reshape(B, H, L, L), loss[0, 0]


# speedup vs baseline: 5.0231x; 5.0231x over previous
import functools

import jax
import jax.numpy as jnp
from jax.experimental import pallas as pl
from jax.experimental.pallas import tpu as pltpu

B, H, L, E = 32, 16, 192, 3
BH = B * H
TOP_P = 0.5
NOISE_EPS = 0.01
EPS_C = 1e-10
BLK = 8
GRID = BH // BLK


def _moe_body(x_ref, eps_ref, w_ref, m_ref, out_ref, loss_ref, acc_ref):
    step = pl.program_id(0)

    @pl.when(step == 0)
    def _init():
        acc_ref[...] = jnp.zeros_like(acc_ref)

    m0 = m_ref[0]
    m1 = m_ref[1]
    m2 = m_ref[2]
    rows = jax.lax.broadcasted_iota(jnp.int32, (L, L), 0)
    cols = jax.lax.broadcasted_iota(jnp.int32, (L, L), 1)
    eye = jnp.where(rows == cols, 1.0, 0.0)

    s0_tot = jnp.zeros((1, L), jnp.float32)
    s1_tot = jnp.zeros((1, L), jnp.float32)
    ent_tot = jnp.zeros((1, L), jnp.float32)
    zero = jnp.zeros((1, L), jnp.float32)
    one = jnp.ones((1, L), jnp.float32)

    for i in range(BLK):
        a = x_ref[i]
        res = jnp.dot(a, w_ref[...], preferred_element_type=jnp.float32)
        t = jnp.transpose(res)
        ee = eps_ref[i]
        n0 = t[0:1, :] + ee[0:1, :] * (jax.nn.softplus(t[3:4, :]) + NOISE_EPS)
        n1 = t[1:2, :] + ee[1:2, :] * (jax.nn.softplus(t[4:5, :]) + NOISE_EPS)
        n2 = t[2:3, :] + ee[2:3, :] * (jax.nn.softplus(t[5:6, :]) + NOISE_EPS)
        mx = jnp.maximum(jnp.maximum(n0, n1), n2)
        x0 = jnp.exp(n0 - mx)
        x1 = jnp.exp(n1 - mx)
        x2 = jnp.exp(n2 - mx)
        rz = 1.0 / (x0 + x1 + x2)
        p0 = x0 * rz
        p1 = x1 * rz
        p2 = x2 * rz
        ent_tot += (p0 * jnp.log(p0 + EPS_C) + p1 * jnp.log(p1 + EPS_C)
                    + p2 * jnp.log(p2 + EPS_C))
        a10 = p1 > p0
        a20 = p2 > p0
        a01 = p0 >= p1
        a21 = p2 > p1
        a02 = p0 >= p2
        a12 = p1 >= p2
        cb0 = jnp.where(a10, p1, zero) + jnp.where(a20, p2, zero)
        cb1 = jnp.where(a01, p0, zero) + jnp.where(a21, p2, zero)
        cb2 = jnp.where(a02, p0, zero) + jnp.where(a12, p1, zero)
        k0 = cb0 <= TOP_P
        k1 = cb1 <= TOP_P
        k2 = cb2 <= TOP_P
        r0 = a10.astype(jnp.int32) + a20.astype(jnp.int32)
        r1 = a01.astype(jnp.int32) + a21.astype(jnp.int32)
        r2 = a02.astype(jnp.int32) + a12.astype(jnp.int32)
        s0_tot += (jnp.where(r0 == 0, p0, zero) + jnp.where(r1 == 0, p1, zero)
                   + jnp.where(r2 == 0, p2, zero))
        s1_tot += (jnp.where((r0 == 1) & k0, p0, zero)
                   + jnp.where((r1 == 1) & k1, p1, zero)
                   + jnp.where((r2 == 1) & k2, p2, zero))
        kmat = jnp.concatenate(
            [jnp.where(k0, one, zero), jnp.where(k1, one, zero),
             jnp.where(k2, one, zero)], axis=0)
        kt = jnp.transpose(kmat)
        out_ref[i] = (kt[:, 0:1] * m0 + kt[:, 1:2] * m1 + kt[:, 2:3] * m2
                      + eye)

    acc_ref[0:1, :] += s0_tot
    acc_ref[1:2, :] += s1_tot
    acc_ref[2:3, :] += ent_tot

    @pl.when(step == GRID - 1)
    def _finalize():
        s0 = acc_ref[0:1, :]
        s1 = acc_ref[1:2, :]
        n = float(L * E)
        tot = jnp.sum(s0) + jnp.sum(s1)
        sq = jnp.sum(s0 * s0) + jnp.sum(s1 * s1)
        mean = tot / n
        var = (sq - n * mean * mean) / (n - 1.0)
        loss_imp = var / (mean * mean + EPS_C)
        loss_dyn = -jnp.sum(acc_ref[2:3, :]) / float(BH * E)
        loss_ref[...] = jnp.reshape(loss_imp + 0.1 * loss_dyn, (1, 1))


@functools.partial(jax.jit, static_argnames=())
def kernel(x, masks, W_gate, W_noise):
    xf = x.reshape(BH, L, L)
    eps = jax.random.normal(jax.random.key(42), (BH, L, E), dtype=jnp.float32)
    eps_t = jnp.transpose(eps, (0, 2, 1))
    w = jnp.concatenate(
        [W_gate, W_noise, jnp.zeros((2, L), jnp.float32)], axis=0).T
    masks_t = jnp.transpose(masks, (1, 0, 2))
    out, loss = pl.pallas_call(
        _moe_body,
        grid=(GRID,),
        in_specs=[
            pl.BlockSpec((BLK, L, L), lambda i: (i, 0, 0)),
            pl.BlockSpec((BLK, E, L), lambda i: (i, 0, 0)),
            pl.BlockSpec((L, 8), lambda i: (0, 0)),
            pl.BlockSpec((E, L, L), lambda i: (0, 0, 0)),
        ],
        out_specs=[
            pl.BlockSpec((BLK, L, L), lambda i: (i, 0, 0)),
            pl.BlockSpec((1, 1), lambda i: (0, 0)),
        ],
        out_shape=[
            jax.ShapeDtypeStruct((BH, L, L), jnp.float32),
            jax.ShapeDtypeStruct((1, 1), jnp.float32),
        ],
        scratch_shapes=[
            pltpu.VMEM((8, L), jnp.float32),
        ],
        compiler_params=pltpu.CompilerParams(
            dimension_semantics=("arbitrary",),
        ),
    )(xf, eps_t, w, masks_t)
    return out.reshape(B, H, L, L), loss[0, 0]


# speedup vs baseline: 5.1673x; 1.0287x over previous
import functools

import jax
import jax.numpy as jnp
from jax.experimental import pallas as pl
from jax.experimental.pallas import tpu as pltpu

B, H, L, E = 32, 16, 192, 3
BH = B * H
TOP_P = 0.5
NOISE_EPS = 0.01
EPS_C = 1e-10
BLK = 16
GRID = BH // BLK


def _moe_body(x_ref, eps_ref, w_ref, m_ref, out_ref, loss_ref, acc_ref):
    step = pl.program_id(0)

    @pl.when(step == 0)
    def _init():
        acc_ref[...] = jnp.zeros_like(acc_ref)

    m0 = m_ref[0]
    m1 = m_ref[1]
    m2 = m_ref[2]
    rows = jax.lax.broadcasted_iota(jnp.int32, (L, L), 0)
    cols = jax.lax.broadcasted_iota(jnp.int32, (L, L), 1)
    eye = jnp.where(rows == cols, 1.0, 0.0)

    s0_tot = jnp.zeros((1, L), jnp.float32)
    s1_tot = jnp.zeros((1, L), jnp.float32)
    ent_tot = jnp.zeros((1, L), jnp.float32)
    zero = jnp.zeros((1, L), jnp.float32)
    one = jnp.ones((1, L), jnp.float32)

    for i in range(BLK):
        a = x_ref[i]
        res = jnp.dot(a, w_ref[...], preferred_element_type=jnp.float32)
        t = jnp.transpose(res)
        ee = eps_ref[i]
        n0 = t[0:1, :] + ee[0:1, :] * (jax.nn.softplus(t[3:4, :]) + NOISE_EPS)
        n1 = t[1:2, :] + ee[1:2, :] * (jax.nn.softplus(t[4:5, :]) + NOISE_EPS)
        n2 = t[2:3, :] + ee[2:3, :] * (jax.nn.softplus(t[5:6, :]) + NOISE_EPS)
        mx = jnp.maximum(jnp.maximum(n0, n1), n2)
        x0 = jnp.exp(n0 - mx)
        x1 = jnp.exp(n1 - mx)
        x2 = jnp.exp(n2 - mx)
        rz = 1.0 / (x0 + x1 + x2)
        p0 = x0 * rz
        p1 = x1 * rz
        p2 = x2 * rz
        ent_tot += (p0 * jnp.log(p0 + EPS_C) + p1 * jnp.log(p1 + EPS_C)
                    + p2 * jnp.log(p2 + EPS_C))
        a10 = p1 > p0
        a20 = p2 > p0
        a01 = p0 >= p1
        a21 = p2 > p1
        a02 = p0 >= p2
        a12 = p1 >= p2
        cb0 = jnp.where(a10, p1, zero) + jnp.where(a20, p2, zero)
        cb1 = jnp.where(a01, p0, zero) + jnp.where(a21, p2, zero)
        cb2 = jnp.where(a02, p0, zero) + jnp.where(a12, p1, zero)
        k0 = cb0 <= TOP_P
        k1 = cb1 <= TOP_P
        k2 = cb2 <= TOP_P
        r0 = a10.astype(jnp.int32) + a20.astype(jnp.int32)
        r1 = a01.astype(jnp.int32) + a21.astype(jnp.int32)
        r2 = a02.astype(jnp.int32) + a12.astype(jnp.int32)
        s0_tot += (jnp.where(r0 == 0, p0, zero) + jnp.where(r1 == 0, p1, zero)
                   + jnp.where(r2 == 0, p2, zero))
        s1_tot += (jnp.where((r0 == 1) & k0, p0, zero)
                   + jnp.where((r1 == 1) & k1, p1, zero)
                   + jnp.where((r2 == 1) & k2, p2, zero))
        kmat = jnp.concatenate(
            [jnp.where(k0, one, zero), jnp.where(k1, one, zero),
             jnp.where(k2, one, zero)], axis=0)
        kt = jnp.transpose(kmat)
        out_ref[i] = (kt[:, 0:1] * m0 + kt[:, 1:2] * m1 + kt[:, 2:3] * m2
                      + eye)

    acc_ref[0:1, :] += s0_tot
    acc_ref[1:2, :] += s1_tot
    acc_ref[2:3, :] += ent_tot

    @pl.when(step == GRID - 1)
    def _finalize():
        s0 = acc_ref[0:1, :]
        s1 = acc_ref[1:2, :]
        n = float(L * E)
        tot = jnp.sum(s0) + jnp.sum(s1)
        sq = jnp.sum(s0 * s0) + jnp.sum(s1 * s1)
        mean = tot / n
        var = (sq - n * mean * mean) / (n - 1.0)
        loss_imp = var / (mean * mean + EPS_C)
        loss_dyn = -jnp.sum(acc_ref[2:3, :]) / float(BH * E)
        loss_ref[...] = jnp.reshape(loss_imp + 0.1 * loss_dyn, (1, 1))


@functools.partial(jax.jit, static_argnames=())
def kernel(x, masks, W_gate, W_noise):
    xf = x.reshape(BH, L, L)
    eps = jax.random.normal(jax.random.key(42), (BH, L, E), dtype=jnp.float32)
    eps_t = jnp.transpose(eps, (0, 2, 1))
    w = jnp.concatenate(
        [W_gate, W_noise, jnp.zeros((2, L), jnp.float32)], axis=0).T
    masks_t = jnp.transpose(masks, (1, 0, 2))
    out, loss = pl.pallas_call(
        _moe_body,
        grid=(GRID,),
        in_specs=[
            pl.BlockSpec((BLK, L, L), lambda i: (i, 0, 0)),
            pl.BlockSpec((BLK, E, L), lambda i: (i, 0, 0)),
            pl.BlockSpec((L, 8), lambda i: (0, 0)),
            pl.BlockSpec((E, L, L), lambda i: (0, 0, 0)),
        ],
        out_specs=[
            pl.BlockSpec((BLK, L, L), lambda i: (i, 0, 0)),
            pl.BlockSpec((1, 1), lambda i: (0, 0)),
        ],
        out_shape=[
            jax.ShapeDtypeStruct((BH, L, L), jnp.float32),
            jax.ShapeDtypeStruct((1, 1), jnp.float32),
        ],
        scratch_shapes=[
            pltpu.VMEM((8, L), jnp.float32),
        ],
        compiler_params=pltpu.CompilerParams(
            dimension_semantics=("arbitrary",),
        ),
    )(xf, eps_t, w, masks_t)
    return out.reshape(B, H, L, L), loss[0, 0]
